# baseline (speedup 1.0000x reference)
import functools

import jax
import jax.numpy as jnp
from jax import lax
from jax.experimental import pallas as pl
from jax.experimental.pallas import tpu as pltpu
from jax.experimental.pallas import tpu_sc as plsc

VOC = 1000000
D = 64
B = 16384
SEQ = 5
H = 128

NC = 2
NS = 16
NW = NC * NS
N_IDX = B * SEQ
ROWS_PER_W = N_IDX // NW
CHUNK = 128
NCHUNK = ROWS_PER_W // CHUNK

TW = 512
NTB = 977
H2 = TW * NTB


def _pair_table(ET):
    def body(a_ref, b_ref, o_ref):
        o_ref[:, :D] = a_ref[...].T
        o_ref[:, D:] = b_ref[...].T

    return pl.pallas_call(
        body,
        grid=(NTB,),
        in_specs=[pl.BlockSpec((D, TW), lambda i: (0, i)),
                  pl.BlockSpec((D, TW), lambda i: (0, i + NTB))],
        out_specs=pl.BlockSpec((TW, 2 * D), lambda i: (i, 0)),
        out_shape=jax.ShapeDtypeStruct((H2, 2 * D), jnp.float32),
    )(ET, ET)


def _sc_gather(idx_3d, table2):
    mesh = plsc.VectorSubcoreMesh(core_axis_name="c", subcore_axis_name="s")

    @functools.partial(
        pl.kernel,
        out_type=jax.ShapeDtypeStruct((N_IDX, 2 * D), jnp.float32),
        mesh=mesh,
        scratch_types=[
            pltpu.VMEM((NCHUNK, CHUNK), jnp.int32),
            pltpu.VMEM((2, CHUNK, 2 * D), jnp.float32),
            pltpu.SemaphoreType.DMA((2,)),
            pltpu.SemaphoreType.DMA((2,)),
        ],
        compiler_params=pltpu.CompilerParams(use_tc_tiling_on_sc=False),
    )
    def gather_kernel(idx_hbm, table_hbm, out_hbm, idx_v, rows_v, gsem, osem):
        wid = lax.axis_index("s") * NC + lax.axis_index("c")
        base = wid * ROWS_PER_W
        pltpu.sync_copy(idx_hbm.at[wid], idx_v)

        def start_gather(j, b):
            return pltpu.async_copy(
                table_hbm.at[idx_v.at[j]], rows_v.at[b], gsem.at[b]
            )

        def start_out(j, b):
            return pltpu.async_copy(
                rows_v.at[b], out_hbm.at[pl.ds(base + j * CHUNK, CHUNK)],
                osem.at[b],
            )

        copies = {}
        copies[("g", 0)] = start_gather(0, 0)
        for j in range(NCHUNK):
            b = j % 2
            if j + 1 < NCHUNK:
                b2 = (j + 1) % 2
                if j >= 1:
                    copies[("o", j - 1)].wait()
                copies[("g", j + 1)] = start_gather(j + 1, b2)
            copies[("g", j)].wait()
            copies[("o", j)] = start_out(j, b)
        copies[("o", NCHUNK - 2)].wait()
        copies[("o", NCHUNK - 1)].wait()

    return gather_kernel(idx_3d, table2)


def _mlp_body(emb_ref, hs_ref, w1cat_ref, b1cat_ref, w2blk_ref, b2cat_ref,
              synt0_ref, sent0_ref, synt1_ref, sent1_ref, synt2_ref, sent2_ref):
    def sel(s):
        pair = emb_ref[:, s * 2 * D:(s + 1) * 2 * D]
        h = hs_ref[:, s:s + 1]
        return jnp.where(h > 0.5, pair[:, D:], pair[:, :D])

    g01 = jnp.concatenate([sel(0), sel(1)], axis=1)
    p = jnp.dot(g01, w1cat_ref[:2 * D, :], preferred_element_type=jnp.float32) \
        + b1cat_ref[0, :]
    w1c = w1cat_ref[2 * D:, :]
    souts = (synt0_ref, synt1_ref, synt2_ref)
    nouts = (sent0_ref, sent1_ref, sent2_ref)
    for k in range(3):
        ek = sel(2 + k)
        h = jnp.clip(p + jnp.dot(ek, w1c, preferred_element_type=jnp.float32),
                     -1.0, 1.0)
        lg = jnp.dot(h, w2blk_ref[...], preferred_element_type=jnp.float32) \
            + b2cat_ref[0, :]
        souts[k][...] = lg[:, 0:1]
        l0, l1 = lg[:, 1:2], lg[:, 2:3]
        m = jnp.maximum(l0, l1)
        e0 = jnp.exp(l0 - m)
        e1 = jnp.exp(l1 - m)
        inv = 1.0 / (e0 + e1)
        nouts[k][...] = jnp.concatenate([e0 * inv, e1 * inv], axis=1)


def _tc_score(emb2, halfsel, w1_synt, b1_synt, w2_synt, b2_synt,
              w1_sent, b1_sent, w2_sent, b2_sent, block_b=2048,
              interpret=False):
    w1cat = jnp.concatenate([w1_synt, w1_sent], axis=1)
    b1cat = jnp.concatenate([b1_synt, b1_sent]).reshape(1, 2 * H)
    w2blk = jnp.zeros((2 * H, 8), jnp.float32)
    w2blk = w2blk.at[:H, 0:1].set(w2_synt)
    w2blk = w2blk.at[H:, 1:3].set(w2_sent)
    b2cat = jnp.zeros((1, 8), jnp.float32)
    b2cat = b2cat.at[0, 0].set(b2_synt[0])
    b2cat = b2cat.at[0, 1].set(b2_sent[0])
    b2cat = b2cat.at[0, 2].set(b2_sent[1])
    grid = (B // block_b,)
    full = lambda shape: pl.BlockSpec(shape, lambda i: (0, 0))
    row = lambda w: pl.BlockSpec((block_b, w), lambda i: (i, 0))
    out_sd = [jax.ShapeDtypeStruct((B, 1), jnp.float32),
              jax.ShapeDtypeStruct((B, 2), jnp.float32)] * 3
    out_specs = [row(1), row(2)] * 3
    return pl.pallas_call(
        _mlp_body,
        grid=grid,
        in_specs=[row(SEQ * 2 * D), row(8), full((3 * D, 2 * H)),
                  full((1, 2 * H)), full((2 * H, 8)), full((1, 8))],
        out_specs=out_specs,
        out_shape=out_sd,
        interpret=interpret,
    )(emb2, halfsel, w1cat, b1cat, w2blk, b2cat)


def kernel(x, E, w1_synt, b1_synt, w2_synt, b2_synt,
           w1_sent, b1_sent, w2_sent, b2_sent):
    xi = x.reshape(N_IDX).astype(jnp.int32)
    table2 = _pair_table(E.T)
    half = xi >= H2
    pair_idx = jnp.where(half, xi - H2, xi).reshape(NW, NCHUNK, CHUNK)
    emb = _sc_gather(pair_idx, table2)
    emb2 = emb.reshape(B, SEQ * 2 * D)
    halfsel = jnp.pad(half.reshape(B, SEQ).astype(jnp.float32),
                      ((0, 0), (0, 8 - SEQ)))
    outs = _tc_score(emb2, halfsel, w1_synt, b1_synt, w2_synt, b2_synt,
                     w1_sent, b1_sent, w2_sent, b2_sent)
    return (outs[0], outs[1], outs[2], outs[3], outs[4], outs[5])


# speedup vs baseline: 1.9055x; 1.9055x over previous
import functools

import jax
import jax.numpy as jnp
from jax import lax
from jax.experimental import pallas as pl
from jax.experimental.pallas import tpu as pltpu
from jax.experimental.pallas import tpu_sc as plsc

VOC = 1000000
D = 64
B = 16384
SEQ = 5
H = 128

NC = 2
NS = 16
NW = NC * NS
N_IDX = B * SEQ
ROWS_PER_W = N_IDX // NW
CHUNK = 128
NCHUNK = ROWS_PER_W // CHUNK

TW = 4352
NTB = 115
H2 = TW * NTB


def _pair_table(ET):
    def body(a_ref, b_ref, o_ref):
        o_ref[...] = jnp.concatenate([a_ref[...].T, b_ref[...].T], axis=1)

    return pl.pallas_call(
        body,
        grid=(NTB,),
        in_specs=[pl.BlockSpec((D, TW), lambda i: (0, i)),
                  pl.BlockSpec((D, TW), lambda i: (0, i + NTB))],
        out_specs=pl.BlockSpec((TW, 2 * D), lambda i: (i, 0)),
        out_shape=jax.ShapeDtypeStruct((H2, 2 * D), jnp.float32),
    )(ET, ET)


def _sc_gather(idx_3d, table2):
    mesh = plsc.VectorSubcoreMesh(core_axis_name="c", subcore_axis_name="s")

    @functools.partial(
        pl.kernel,
        out_type=jax.ShapeDtypeStruct((N_IDX, 2 * D), jnp.float32),
        mesh=mesh,
        scratch_types=[
            pltpu.VMEM((NCHUNK, CHUNK), jnp.int32),
            pltpu.VMEM((2, CHUNK, 2 * D), jnp.float32),
            pltpu.SemaphoreType.DMA((2,)),
            pltpu.SemaphoreType.DMA((2,)),
        ],
        compiler_params=pltpu.CompilerParams(use_tc_tiling_on_sc=False),
    )
    def gather_kernel(idx_hbm, table_hbm, out_hbm, idx_v, rows_v, gsem, osem):
        wid = lax.axis_index("s") * NC + lax.axis_index("c")
        base = wid * ROWS_PER_W
        pltpu.sync_copy(idx_hbm.at[wid], idx_v)

        def start_gather(j, b):
            return pltpu.async_copy(
                table_hbm.at[idx_v.at[j]], rows_v.at[b], gsem.at[b]
            )

        def start_out(j, b):
            return pltpu.async_copy(
                rows_v.at[b], out_hbm.at[pl.ds(base + j * CHUNK, CHUNK)],
                osem.at[b],
            )

        copies = {}
        copies[("g", 0)] = start_gather(0, 0)
        for j in range(NCHUNK):
            b = j % 2
            if j + 1 < NCHUNK:
                b2 = (j + 1) % 2
                if j >= 1:
                    copies[("o", j - 1)].wait()
                copies[("g", j + 1)] = start_gather(j + 1, b2)
            copies[("g", j)].wait()
            copies[("o", j)] = start_out(j, b)
        copies[("o", NCHUNK - 2)].wait()
        copies[("o", NCHUNK - 1)].wait()

    return gather_kernel(idx_3d, table2)


def _mlp_body(emb_ref, hs_ref, w1cat_ref, b1cat_ref, w2blk_ref, b2cat_ref,
              synt0_ref, sent0_ref, synt1_ref, sent1_ref, synt2_ref, sent2_ref):
    def sel(s):
        pair = emb_ref[:, s * 2 * D:(s + 1) * 2 * D]
        h = hs_ref[:, s:s + 1]
        return jnp.where(h > 0.5, pair[:, D:], pair[:, :D])

    g01 = jnp.concatenate([sel(0), sel(1)], axis=1)
    p = jnp.dot(g01, w1cat_ref[:2 * D, :], preferred_element_type=jnp.float32) \
        + b1cat_ref[0, :]
    w1c = w1cat_ref[2 * D:, :]
    souts = (synt0_ref, synt1_ref, synt2_ref)
    nouts = (sent0_ref, sent1_ref, sent2_ref)
    for k in range(3):
        ek = sel(2 + k)
        h = jnp.clip(p + jnp.dot(ek, w1c, preferred_element_type=jnp.float32),
                     -1.0, 1.0)
        lg = jnp.dot(h, w2blk_ref[...], preferred_element_type=jnp.float32) \
            + b2cat_ref[0, :]
        souts[k][...] = lg[:, 0:1]
        l0, l1 = lg[:, 1:2], lg[:, 2:3]
        m = jnp.maximum(l0, l1)
        e0 = jnp.exp(l0 - m)
        e1 = jnp.exp(l1 - m)
        inv = 1.0 / (e0 + e1)
        nouts[k][...] = jnp.concatenate([e0 * inv, e1 * inv], axis=1)


def _tc_score(emb2, halfsel, w1_synt, b1_synt, w2_synt, b2_synt,
              w1_sent, b1_sent, w2_sent, b2_sent, block_b=2048,
              interpret=False):
    w1cat = jnp.concatenate([w1_synt, w1_sent], axis=1)
    b1cat = jnp.concatenate([b1_synt, b1_sent]).reshape(1, 2 * H)
    w2blk = jnp.zeros((2 * H, 8), jnp.float32)
    w2blk = w2blk.at[:H, 0:1].set(w2_synt)
    w2blk = w2blk.at[H:, 1:3].set(w2_sent)
    b2cat = jnp.zeros((1, 8), jnp.float32)
    b2cat = b2cat.at[0, 0].set(b2_synt[0])
    b2cat = b2cat.at[0, 1].set(b2_sent[0])
    b2cat = b2cat.at[0, 2].set(b2_sent[1])
    grid = (B // block_b,)
    full = lambda shape: pl.BlockSpec(shape, lambda i: (0, 0))
    row = lambda w: pl.BlockSpec((block_b, w), lambda i: (i, 0))
    out_sd = [jax.ShapeDtypeStruct((B, 1), jnp.float32),
              jax.ShapeDtypeStruct((B, 2), jnp.float32)] * 3
    out_specs = [row(1), row(2)] * 3
    return pl.pallas_call(
        _mlp_body,
        grid=grid,
        in_specs=[row(SEQ * 2 * D), row(8), full((3 * D, 2 * H)),
                  full((1, 2 * H)), full((2 * H, 8)), full((1, 8))],
        out_specs=out_specs,
        out_shape=out_sd,
        interpret=interpret,
    )(emb2, halfsel, w1cat, b1cat, w2blk, b2cat)


def kernel(x, E, w1_synt, b1_synt, w2_synt, b2_synt,
           w1_sent, b1_sent, w2_sent, b2_sent):
    xi = x.reshape(N_IDX).astype(jnp.int32)
    table2 = _pair_table(E.T)
    half = xi >= H2
    pair_idx = jnp.where(half, xi - H2, xi).reshape(NW, NCHUNK, CHUNK)
    emb = _sc_gather(pair_idx, table2)
    emb2 = emb.reshape(B, SEQ * 2 * D)
    halfsel = jnp.pad(half.reshape(B, SEQ).astype(jnp.float32),
                      ((0, 0), (0, 8 - SEQ)))
    outs = _tc_score(emb2, halfsel, w1_synt, b1_synt, w2_synt, b2_synt,
                     w1_sent, b1_sent, w2_sent, b2_sent)
    return (outs[0], outs[1], outs[2], outs[3], outs[4], outs[5])
